# baseline (speedup 1.0000x reference)
import functools

import jax
import jax.numpy as jnp
from jax import lax
from jax.experimental import pallas as pl
from jax.experimental.pallas import tpu as pltpu
from jax.experimental.pallas import tpu_sc as plsc

H = 224
W = 224
NPIX = H * W
NPLANE = 24
K = 1000
KPAD = 1024
KBUF = KPAD + 16
CHUNKS = NPIX // 16
import numpy as np
THRESH = float(np.float32(0.3) + np.float32(1e-5))


def _sc_compact_body(pred_hbm, tgt_hbm, idx_hbm, cnt_hbm,
                     pred_v, tgt_v, o0, o1, o2, o3, cnt_v):
  wid = lax.axis_index("s") * 2 + lax.axis_index("c")

  @pl.when(wid < NPLANE)
  def _():
    pltpu.sync_copy(pred_hbm.at[wid], pred_v)
    pltpu.sync_copy(tgt_hbm.at[wid], tgt_v)
    lanes = lax.iota(jnp.int32, 16)

    one = jnp.int32(1)
    zero = jnp.int32(0)

    def masks_at(it):
      base = it * 16
      pv = pred_v[pl.ds(base, 16)]
      tv = tgt_v[pl.ds(base, 16)]
      pbi = jnp.where(jnp.abs(pv - 1.0) <= THRESH, one, zero)
      tbi = jnp.where(jnp.abs(tv - 1.0) <= THRESH, one, zero)
      mi0 = pbi * (1 - tbi)
      mi1 = tbi * (1 - pbi)
      return mi0, mi1, tbi, pbi

    def body(it, carry):
      c0, c1, c2, c3 = carry
      mi0, mi1, tbi, pbi = masks_at(it)
      idxv = it * 16 + lanes
      trash = KPAD + lanes

      def emit(ref, cur, mi):
        ranks = plsc.cumsum(mi)
        n = jnp.sum(mi)
        pos = jnp.minimum(cur + ranks - 1, trash)
        pos = pos * mi + trash * (1 - mi)
        plsc.store_scatter(ref, [pos], idxv)
        return cur + n

      c0 = emit(o0, c0, mi0)
      c1 = emit(o1, c1, mi1)
      c2 = emit(o2, c2, tbi)
      c3 = emit(o3, c3, pbi)
      return c0, c1, c2, c3

    BLK = 16
    z = jnp.int32(0)

    def wcond(carry):
      it, c0, c1, c2, c3 = carry
      full = ((c0 >= KPAD) & (c1 >= KPAD) & (c2 >= KPAD) & (c3 >= KPAD))
      return jnp.logical_and(it < CHUNKS, jnp.logical_not(full))

    def wbody(carry):
      it, c0, c1, c2, c3 = carry

      def inner(j, cs):
        return body(it + j, cs)

      c0, c1, c2, c3 = lax.fori_loop(0, BLK, inner, (c0, c1, c2, c3))
      return it + BLK, c0, c1, c2, c3

    it_end, c0, c1, c2, c3 = lax.while_loop(
        wcond, wbody, (z, z, z, z, z))

    def body2(it, vecs):
      v0, v1, v2, v3 = vecs
      mi0, mi1, tbi, pbi = masks_at(it)
      return v0 + mi0, v1 + mi1, v2 + tbi, v3 + pbi

    zv = jnp.zeros((16,), jnp.int32)
    v0, v1, v2, v3 = lax.fori_loop(it_end, CHUNKS, body2, (zv, zv, zv, zv))
    c0 = c0 + jnp.sum(v0)
    c1 = c1 + jnp.sum(v1)
    c2 = c2 + jnp.sum(v2)
    c3 = c3 + jnp.sum(v3)

    cvec = jnp.where(lanes == 0, jnp.full((16,), c0, jnp.int32),
           jnp.where(lanes == 1, jnp.full((16,), c1, jnp.int32),
           jnp.where(lanes == 2, jnp.full((16,), c2, jnp.int32),
           jnp.where(lanes == 3, jnp.full((16,), c3, jnp.int32),
                     jnp.zeros((16,), jnp.int32)))))
    cnt_v[...] = cvec
    pltpu.sync_copy(cnt_v, cnt_hbm.at[wid])
    pltpu.sync_copy(o0.at[pl.ds(0, KPAD)], idx_hbm.at[wid, 0])
    pltpu.sync_copy(o1.at[pl.ds(0, KPAD)], idx_hbm.at[wid, 1])
    pltpu.sync_copy(o2.at[pl.ds(0, KPAD)], idx_hbm.at[wid, 2])
    pltpu.sync_copy(o3.at[pl.ds(0, KPAD)], idx_hbm.at[wid, 3])


def _sc_compact(pred24, tgt24):
  f = pl.kernel(
      _sc_compact_body,
      out_type=(
          jax.ShapeDtypeStruct((NPLANE, 4, KPAD), jnp.int32),
          jax.ShapeDtypeStruct((NPLANE, 16), jnp.int32),
      ),
      mesh=plsc.VectorSubcoreMesh(core_axis_name="c", subcore_axis_name="s"),
      compiler_params=pltpu.CompilerParams(needs_layout_passes=False),
      scratch_types=[
          pltpu.VMEM((NPIX,), jnp.float32),
          pltpu.VMEM((NPIX,), jnp.float32),
          pltpu.VMEM((KBUF,), jnp.int32),
          pltpu.VMEM((KBUF,), jnp.int32),
          pltpu.VMEM((KBUF,), jnp.int32),
          pltpu.VMEM((KBUF,), jnp.int32),
          pltpu.VMEM((16,), jnp.int32),
      ],
  )
  return f(pred24, tgt24)


def _tc_pairwise_body(xcol_ref, trow_ref, cnt_ref, out_ref):
  i = pl.program_id(0)

  @pl.when(i == 0)
  def _():
    out_ref[0, 0] = jnp.float32(0.0)

  plane = i // 2
  d = i % 2
  nx = cnt_ref[plane, d]
  ny = cnt_ref[plane, 2 + d]
  nx_eff = jnp.minimum(nx, K)
  ny_eff = jnp.minimum(ny, K)

  ti = trow_ref[0]
  lanes = lax.broadcasted_iota(jnp.int32, (1, 128), 1)
  inf = jnp.float32(jnp.inf)
  trs = []
  tcs = []
  for tj in range(8):
    trow = ti[tj:tj + 1, :]
    tvalid = (tj * 128 + lanes) < ny_eff
    trs.append((trow // W).astype(jnp.float32))
    tcs.append(jnp.where(tvalid, (trow % W).astype(jnp.float32), inf))

  sub = lax.broadcasted_iota(jnp.int32, (128, 1), 0)
  part = jnp.float32(0.0)
  for pj in range(8):
    xi = xcol_ref[0, pl.ds(pj * 128, 128), :]
    pr = (xi // W).astype(jnp.float32)
    pc = (xi % W).astype(jnp.float32)
    md2 = jnp.full((128, 128), inf, jnp.float32)
    for tj in range(8):
      dr = pr - trs[tj]
      dc = pc - tcs[tj]
      md2 = jnp.minimum(md2, dr * dr + dc * dc)
    mind = jnp.sqrt(jnp.min(md2, axis=1, keepdims=True))
    pslot = pj * 128 + sub
    part += jnp.sum(jnp.where(pslot < nx_eff, mind, jnp.float32(0.0)))

  gate = jnp.logical_and(nx > 0, ny > 0)
  contrib = jnp.where(gate, part / ny.astype(jnp.float32), jnp.float32(0.0))
  out_ref[0, 0] += contrib / jnp.float32(2 * NPLANE)


def _tc_pairwise(xcol, trow, cnt):
  return pl.pallas_call(
      _tc_pairwise_body,
      grid=(48,),
      in_specs=[
          pl.BlockSpec((1, KPAD, 1),
                       lambda i: (4 * (i // 2) + i % 2, 0, 0)),
          pl.BlockSpec((1, 8, 128),
                       lambda i: (4 * (i // 2) + 2 + i % 2, 0, 0)),
          pl.BlockSpec(memory_space=pltpu.SMEM),
      ],
      out_specs=pl.BlockSpec((1, 1), lambda i: (0, 0),
                             memory_space=pltpu.SMEM),
      out_shape=jax.ShapeDtypeStruct((1, 1), jnp.float32),
  )(xcol, trow, cnt)


@jax.jit
def kernel(pred, target):
  pred24 = pred.reshape(NPLANE, NPIX)
  tgt24 = target.reshape(NPLANE, NPIX)
  idx, cnt = _sc_compact(pred24, tgt24)
  xcol = idx.reshape(NPLANE * 4, KPAD, 1)
  trow = idx.reshape(NPLANE * 4, 8, 128)
  out = _tc_pairwise(xcol, trow, cnt)
  return out[0, 0]


# speedup vs baseline: 1.1308x; 1.1308x over previous
import functools

import jax
import jax.numpy as jnp
from jax import lax
from jax.experimental import pallas as pl
from jax.experimental.pallas import tpu as pltpu
from jax.experimental.pallas import tpu_sc as plsc

H = 224
W = 224
NPIX = H * W
NPLANE = 24
K = 1000
KPAD = 1024
KBUF = KPAD + 16
CHUNKS = NPIX // 16
import numpy as np
THRESH = float(np.float32(0.3) + np.float32(1e-5))


def _sc_compact_body(pred_hbm, tgt_hbm, idx_hbm, cnt_hbm,
                     pred_v, tgt_v, o0, o1, o2, o3, cnt_v):
  wid = lax.axis_index("s") * 2 + lax.axis_index("c")

  @pl.when(wid < NPLANE)
  def _():
    pltpu.sync_copy(pred_hbm.at[wid], pred_v)
    pltpu.sync_copy(tgt_hbm.at[wid], tgt_v)
    lanes = lax.iota(jnp.int32, 16)

    one = jnp.int32(1)
    zero = jnp.int32(0)

    def masks_at(it):
      base = it * 16
      pv = pred_v[pl.ds(base, 16)]
      tv = tgt_v[pl.ds(base, 16)]
      pbi = jnp.where(jnp.abs(pv - 1.0) <= THRESH, one, zero)
      tbi = jnp.where(jnp.abs(tv - 1.0) <= THRESH, one, zero)
      mi0 = pbi * (1 - tbi)
      mi1 = tbi * (1 - pbi)
      return mi0, mi1, tbi, pbi

    def body(it, carry):
      c0, c1, c2, c3 = carry
      mi0, mi1, tbi, pbi = masks_at(it)
      idxv = it * 16 + lanes
      trash = KPAD + lanes

      def emit(ref, cur, mi):
        ranks = plsc.cumsum(mi)
        n = jnp.sum(mi)
        pos = jnp.minimum(cur + ranks - 1, trash)
        pos = pos * mi + trash * (1 - mi)
        plsc.store_scatter(ref, [pos], idxv)
        return cur + n

      c0 = emit(o0, c0, mi0)
      c1 = emit(o1, c1, mi1)
      c2 = emit(o2, c2, tbi)
      c3 = emit(o3, c3, pbi)
      return c0, c1, c2, c3

    BLK = 16
    z = jnp.int32(0)

    def wcond(carry):
      it, c0, c1, c2, c3 = carry
      full = ((c0 >= KPAD) & (c1 >= KPAD) & (c2 >= KPAD) & (c3 >= KPAD))
      return jnp.logical_and(it < CHUNKS, jnp.logical_not(full))

    def wbody(carry):
      it, c0, c1, c2, c3 = carry

      def inner(j, cs):
        return body(it + j, cs)

      c0, c1, c2, c3 = lax.fori_loop(0, BLK, inner, (c0, c1, c2, c3))
      return it + BLK, c0, c1, c2, c3

    it_end, c0, c1, c2, c3 = lax.while_loop(
        wcond, wbody, (z, z, z, z, z))

    def body2(it, vecs):
      v0, v1, v2, v3 = vecs
      mi0, mi1, tbi, pbi = masks_at(it)
      return v0 + mi0, v1 + mi1, v2 + tbi, v3 + pbi

    zv = jnp.zeros((16,), jnp.int32)
    v0, v1, v2, v3 = lax.fori_loop(it_end, CHUNKS, body2, (zv, zv, zv, zv))
    c0 = c0 + jnp.sum(v0)
    c1 = c1 + jnp.sum(v1)
    c2 = c2 + jnp.sum(v2)
    c3 = c3 + jnp.sum(v3)

    cvec = jnp.where(lanes == 0, jnp.full((16,), c0, jnp.int32),
           jnp.where(lanes == 1, jnp.full((16,), c1, jnp.int32),
           jnp.where(lanes == 2, jnp.full((16,), c2, jnp.int32),
           jnp.where(lanes == 3, jnp.full((16,), c3, jnp.int32),
                     jnp.zeros((16,), jnp.int32)))))
    cnt_v[...] = cvec
    pltpu.sync_copy(cnt_v, cnt_hbm.at[wid])
    pltpu.sync_copy(o0.at[pl.ds(0, KPAD)], idx_hbm.at[wid, 0])
    pltpu.sync_copy(o1.at[pl.ds(0, KPAD)], idx_hbm.at[wid, 1])
    pltpu.sync_copy(o2.at[pl.ds(0, KPAD)], idx_hbm.at[wid, 2])
    pltpu.sync_copy(o3.at[pl.ds(0, KPAD)], idx_hbm.at[wid, 3])


def _sc_compact(pred24, tgt24):
  f = pl.kernel(
      _sc_compact_body,
      out_type=(
          jax.ShapeDtypeStruct((NPLANE, 4, KPAD), jnp.int32),
          jax.ShapeDtypeStruct((NPLANE, 16), jnp.int32),
      ),
      mesh=plsc.VectorSubcoreMesh(core_axis_name="c", subcore_axis_name="s"),
      compiler_params=pltpu.CompilerParams(needs_layout_passes=False),
      scratch_types=[
          pltpu.VMEM((NPIX,), jnp.float32),
          pltpu.VMEM((NPIX,), jnp.float32),
          pltpu.VMEM((KBUF,), jnp.int32),
          pltpu.VMEM((KBUF,), jnp.int32),
          pltpu.VMEM((KBUF,), jnp.int32),
          pltpu.VMEM((KBUF,), jnp.int32),
          pltpu.VMEM((16,), jnp.int32),
      ],
  )
  return f(pred24, tgt24)


def _tc_pairwise_body(xcol_ref, trow_ref, nx_ref, ny_ref, out_ref):
  i = pl.program_id(0)

  @pl.when(i == 0)
  def _():
    out_ref[0, 0] = jnp.float32(0.0)

  nx = nx_ref[i]
  ny = ny_ref[i]
  nx_eff = jnp.minimum(nx, K)
  ny_eff = jnp.minimum(ny, K)

  ti = trow_ref[0]
  lanes = lax.broadcasted_iota(jnp.int32, (1, 128), 1)
  inf = jnp.float32(jnp.inf)
  trs = []
  tcs = []
  for tj in range(8):
    trow = ti[tj:tj + 1, :]
    tvalid = (tj * 128 + lanes) < ny_eff
    trs.append((trow // W).astype(jnp.float32))
    tcs.append(jnp.where(tvalid, (trow % W).astype(jnp.float32), inf))

  sub = lax.broadcasted_iota(jnp.int32, (128, 1), 0)
  part = jnp.float32(0.0)
  for pj in range(8):
    xi = xcol_ref[0, pl.ds(pj * 128, 128), :]
    pr = (xi // W).astype(jnp.float32)
    pc = (xi % W).astype(jnp.float32)
    md2 = jnp.full((128, 128), inf, jnp.float32)
    for tj in range(8):
      dr = pr - trs[tj]
      dc = pc - tcs[tj]
      md2 = jnp.minimum(md2, dr * dr + dc * dc)
    mind = jnp.sqrt(jnp.min(md2, axis=1, keepdims=True))
    pslot = pj * 128 + sub
    part += jnp.sum(jnp.where(pslot < nx_eff, mind, jnp.float32(0.0)))

  gate = jnp.logical_and(nx > 0, ny > 0)
  contrib = jnp.where(gate, part / ny.astype(jnp.float32), jnp.float32(0.0))
  out_ref[0, 0] += contrib / jnp.float32(2 * NPLANE)


def _tc_pairwise(xcol, trow, nx, ny):
  return pl.pallas_call(
      _tc_pairwise_body,
      grid=(48,),
      in_specs=[
          pl.BlockSpec((1, KPAD, 1), lambda i: (i, 0, 0)),
          pl.BlockSpec((1, 8, 128), lambda i: (i, 0, 0)),
          pl.BlockSpec(memory_space=pltpu.SMEM),
          pl.BlockSpec(memory_space=pltpu.SMEM),
      ],
      out_specs=pl.BlockSpec((1, 1), lambda i: (0, 0),
                             memory_space=pltpu.SMEM),
      out_shape=jax.ShapeDtypeStruct((1, 1), jnp.float32),
  )(xcol, trow, nx, ny)


@jax.jit
def kernel(pred, target):
  pred24 = pred.reshape(NPLANE, NPIX)
  tgt24 = target.reshape(NPLANE, NPIX)
  idx, cnt = _sc_compact(pred24, tgt24)
  xcol = idx[:, 0:2, :].reshape(48, KPAD, 1)
  y_idx = idx[:, 2:4, :].reshape(48, 8, 128)
  nx = cnt[:, 0:2].reshape(48)
  ny = cnt[:, 2:4].reshape(48)
  out = _tc_pairwise(xcol, y_idx, nx, ny)
  return out[0, 0]
